# baseline (speedup 1.0000x reference)
import functools

import jax
import jax.numpy as jnp
from jax import lax
from jax.experimental import pallas as pl
from jax.experimental.pallas import tpu as pltpu
from jax.experimental.pallas import tpu_sc as plsc

N = 10000
E = 320000
NNFEAT = 128
H = 256
OUT = 1024
NUM_GRAPHS = 16

NC = 2
NS = 16
NW = NC * NS
L = 16

K = 80
EPW = E // NW
EPS = E // NS
NPS = N // NS
HC = 128
WR = 624

_MESH = plsc.VectorSubcoreMesh(core_axis_name="c", subcore_axis_name="s")


def _dosd_body(tab_hbm, fidx_hbm, out_hbm, fidx0, fidx1, vbuf0, vbuf1,
               isem0, isem1, gsem0, gsem1):
    c = lax.axis_index("c")
    s = lax.axis_index("s")
    wid = s * NC + c
    base0 = wid * EPW
    sets = ((fidx0, vbuf0, isem0, gsem0), (fidx1, vbuf1, isem1, gsem1))

    def issue_idx(j, p):
        fi, _, ise, _ = sets[p]
        pltpu.async_copy(fidx_hbm.at[pl.ds(base0 + j * K, K)], fi, ise)

    def issue_gather(p):
        fi, vb, ise, gse = sets[p]
        pltpu.make_async_copy(fidx_hbm.at[pl.ds(0, K)], fi, ise).wait()
        pltpu.async_copy(tab_hbm.at[fi], vb, gse)

    def drain(j, p):
        fi, vb, _, gse = sets[p]
        pltpu.make_async_copy(tab_hbm.at[fi], vb, gse).wait()
        pltpu.sync_copy(vb, out_hbm.at[pl.ds(base0 + j * K, K)])

    issue_idx(0, 0)
    issue_idx(1, 1)
    issue_gather(0)

    def pair(t, _):
        j = t * 2

        def half(j, p):
            pl.when(j + 1 < EPW // K)(lambda: issue_gather(1 - p))
            drain(j, p)
            pl.when(j + 2 < EPW // K)(lambda: issue_idx(j + 2, p))

        half(j, 0)
        half(j + 1, 1)
        return 0

    lax.fori_loop(0, EPW // (2 * K), pair, 0)
    if (EPW // K) % 2:
        drain(EPW // K - 1, 0)


_dosd_gather = functools.partial(
    pl.kernel,
    out_type=jax.ShapeDtypeStruct((E,), jnp.float32),
    mesh=_MESH,
    scratch_types=[
        pltpu.VMEM((K,), jnp.int32),
        pltpu.VMEM((K,), jnp.int32),
        pltpu.VMEM((K,), jnp.float32),
        pltpu.VMEM((K,), jnp.float32),
        pltpu.SemaphoreType.DMA,
        pltpu.SemaphoreType.DMA,
        pltpu.SemaphoreType.DMA,
        pltpu.SemaphoreType.DMA,
    ],
)(_dosd_body)


def _edge_body(esplit, h_hbm, eaw_hbm, src_hbm, dst_hbm, out_hbm,
               sidx0, didx0, gidx0, dscat0, hbuf0, ebuf0,
               sidx1, didx1, gidx1, dscat1, hbuf1, ebuf1,
               aggr_sh,
               isem0, gsem0, esem0, ssem0, isem1, gsem1, esem1, ssem1):
    c = lax.axis_index("c")
    s = lax.axis_index("s")

    def zrow(r, _):
        for cc in range(HC // L):
            ebuf0[r, pl.ds(cc * L, L)] = jnp.zeros((L,), jnp.float32)
        return 0

    lax.fori_loop(0, K, zrow, 0)

    def zcopy(j, _):
        pltpu.sync_copy(ebuf0, aggr_sh.at[pl.ds(s * NPS + j * K, K)])
        return 0

    lax.fori_loop(0, NPS // K, zcopy, 0)
    pltpu.sync_copy(ebuf0.at[pl.ds(0, NPS - (NPS // K) * K)],
                    aggr_sh.at[pl.ds(s * NPS + (NPS // K) * K,
                                     NPS - (NPS // K) * K)])
    plsc.subcore_barrier()

    if esplit:
        base0 = (c * NS + s) * EPW
        goff = 0
        eoff = 0
        nchunks = EPW // K
    else:
        base0 = s * EPS
        goff = c * N
        eoff = c * E
        nchunks = EPS // K

    sets = ((sidx0, didx0, gidx0, dscat0, hbuf0, ebuf0, isem0, gsem0, esem0,
             ssem0),
            (sidx1, didx1, gidx1, dscat1, hbuf1, ebuf1, isem1, gsem1, esem1,
             ssem1))

    def issue_idx(j, p):
        b = base0 + j * K
        si, di, _, _, _, _, ise, _, _, _ = sets[p]
        pltpu.async_copy(src_hbm.at[pl.ds(b, K)], si, ise)
        pltpu.async_copy(dst_hbm.at[pl.ds(b, K)], di, ise)

    def wait_idx(p):
        si, di, _, _, _, _, ise, _, _, _ = sets[p]
        pltpu.make_async_copy(src_hbm.at[pl.ds(0, K)], si, ise).wait()
        pltpu.make_async_copy(dst_hbm.at[pl.ds(0, K)], di, ise).wait()

    def issue_fetch(j, p):
        b = base0 + j * K
        si, di, gi, dsc, hb, eb, _, gse, ese, _ = sets[p]
        for i in range(K // L):
            sl = pl.ds(i * L, L)
            gi[sl] = si[sl] + goff
            dsc[sl] = di[sl]
        pltpu.async_copy(h_hbm.at[gi], hb, gse)
        pltpu.async_copy(eaw_hbm.at[pl.ds(eoff + b, K)], eb, ese)

    def wait_fetch(p):
        _, _, gi, _, hb, eb, _, gse, ese, _ = sets[p]
        pltpu.make_async_copy(h_hbm.at[gi], hb, gse).wait()
        pltpu.make_async_copy(eaw_hbm.at[pl.ds(0, K)], eb, ese).wait()

    def compute(p):
        hb, eb = sets[p][4], sets[p][5]

        def row(r, _):
            for cc in range(HC // L):
                sl = pl.ds(cc * L, L)
                eb[r, sl] = jnp.maximum(hb[r, sl] + eb[r, sl], 0.0)
            return 0

        lax.fori_loop(0, K, row, 0)

    def issue_scatter(p):
        _, _, _, dsc, _, eb, _, _, _, sse = sets[p]
        pltpu.async_copy(eb, aggr_sh.at[dsc], sse, add=True)

    def wait_scatter(p):
        _, _, _, dsc, _, eb, _, _, _, sse = sets[p]
        pltpu.make_async_copy(eb, aggr_sh.at[dsc], sse).wait()

    def body(j, p, static_last=False):
        pbar = 1 - p

        if not static_last:
            @pl.when(j + 1 < nchunks)
            def _():
                pl.when(j >= 1)(lambda: wait_scatter(pbar))
                wait_idx(pbar)
                issue_fetch(j + 1, pbar)
                pl.when(j + 2 < nchunks)(lambda: issue_idx(j + 2, p))

        wait_fetch(p)
        compute(p)
        issue_scatter(p)

    issue_idx(0, 0)
    issue_idx(1, 1)
    wait_idx(0)
    issue_fetch(0, 0)

    def pair(t, _):
        body(t * 2, 0)
        body(t * 2 + 1, 1)
        return 0

    lax.fori_loop(0, nchunks // 2, pair, 0)
    if nchunks % 2:
        body(nchunks - 1, 0, static_last=True)
    wait_scatter(nchunks % 2)
    wait_scatter((nchunks + 1) % 2)
    plsc.subcore_barrier()
    pltpu.sync_copy(aggr_sh.at[pl.ds(s * WR, WR)],
                    out_hbm.at[pl.ds(c * N + s * WR, WR)])

    @pl.when(s == 0)
    def _():
        pltpu.sync_copy(aggr_sh.at[pl.ds(NS * WR, N - NS * WR)],
                        out_hbm.at[pl.ds(c * N + NS * WR, N - NS * WR)])


@functools.cache
def _edge_stage(esplit):
    return functools.partial(
        pl.kernel,
        out_type=jax.ShapeDtypeStruct((2 * N, HC), jnp.float32),
        mesh=_MESH,
        scratch_types=(
            [pltpu.VMEM((K,), jnp.int32)] * 4
            + [pltpu.VMEM((K, HC), jnp.float32)] * 2
            + [pltpu.VMEM((K,), jnp.int32)] * 4
            + [pltpu.VMEM((K, HC), jnp.float32)] * 2
            + [pltpu.VMEM_SHARED((N, HC), jnp.float32)]
            + [pltpu.SemaphoreType.DMA] * 8
        ),
    )(functools.partial(_edge_body, esplit))


EB = 2000


def _eaw_body(split, ea_ref, dv_ref, w_ref, lw_ref, b_ref, o_ref):
    res = (jnp.dot(ea_ref[...], w_ref[...],
                   preferred_element_type=jnp.float32)
           + dv_ref[...] * lw_ref[...] + b_ref[...])
    if split:
        o_ref[0] = res[:, :HC]
        o_ref[1] = res[:, HC:]
    else:
        o_ref[...] = res


def _eaw(l, edge_attr, dosd_vals, params):
    p = params
    we = p[f"We{l}"]
    cin = we.shape[1]
    split = cin == 2 * HC
    if split:
        out_spec = pl.BlockSpec((2, EB, HC), lambda i: (0, i, 0))
        out_shape = jax.ShapeDtypeStruct((2, E, HC), jnp.float32)
    else:
        out_spec = pl.BlockSpec((EB, HC), lambda i: (i, 0))
        out_shape = jax.ShapeDtypeStruct((E, HC), jnp.float32)
    return pl.pallas_call(
        functools.partial(_eaw_body, split),
        grid=(E // EB,),
        in_specs=[pl.BlockSpec((EB, 17), lambda i: (i, 0)),
                  pl.BlockSpec((EB, 1), lambda i: (i, 0)),
                  pl.BlockSpec((17, cin), lambda i: (0, 0)),
                  pl.BlockSpec((1, cin), lambda i: (0, 0)),
                  pl.BlockSpec((1, cin), lambda i: (0, 0))],
        out_specs=out_spec,
        out_shape=out_shape,
    )(edge_attr, dosd_vals, we[:17], we[17].reshape(1, cin),
      p[f"be{l}"].reshape(1, cin))


NB = 1000


def _node_body(first, last, alo_ref, ahi_ref, hlo_ref, hhi_ref, xa_ref,
               wg_ref, bg_ref, wa_ref, ba_ref, wb_ref, bb_ref, wf_ref,
               bf_ref, o_ref):
    gt = jnp.dot(xa_ref[...], wg_ref[...],
                 preferred_element_type=jnp.float32) + bg_ref[...]
    if first:
        inp = alo_ref[...] + ahi_ref[...] + hlo_ref[...] + gt
    else:
        inp = jnp.concatenate(
            [alo_ref[...] + hlo_ref[...], ahi_ref[...] + hhi_ref[...]],
            axis=1) + gt
    bf = jnp.bfloat16
    t1 = jax.nn.relu(jnp.dot(inp.astype(bf), wa_ref[...].astype(bf),
                             preferred_element_type=jnp.float32) + ba_ref[...])
    t2 = jax.nn.relu(jnp.dot(t1.astype(bf), wb_ref[...].astype(bf),
                             preferred_element_type=jnp.float32) + bb_ref[...])
    h3 = jnp.dot(t2.astype(bf), wf_ref[...].astype(bf),
                 preferred_element_type=jnp.float32) + bf_ref[...]
    if not last:
        h3 = jax.nn.relu(h3)
        o_ref[0] = h3[:, :HC]
        o_ref[1] = h3[:, HC:]
    else:
        o_ref[...] = h3


def _node_mlp(l, aggr_flat, h_flat, xA, params):
    p = params
    cin = NNFEAT if l == 1 else H
    first = l == 1
    hcin = cin if first else cin // 2
    last = l == 3
    nblk = N // NB
    args = [aggr_flat, aggr_flat, h_flat, h_flat,
            xA.reshape(1, 21), p[f"Wg{l}"], p[f"bg{l}"].reshape(1, cin),
            p[f"W{l}a"], p[f"b{l}a"].reshape(1, H),
            p[f"W{l}b"], p[f"b{l}b"].reshape(1, OUT),
            p[f"Wf{l}"], p[f"bf{l}"].reshape(1, H)]
    in_specs = [
        pl.BlockSpec((NB, HC), lambda i: (i, 0)),
        pl.BlockSpec((NB, HC), lambda i: (i + nblk, 0)),
        pl.BlockSpec((NB, hcin), lambda i: (i, 0)),
        pl.BlockSpec((NB, hcin), lambda i: (i, 0) if first
                     else (i + nblk, 0)),
        pl.BlockSpec((1, 21), lambda i: (0, 0)),
        pl.BlockSpec((21, cin), lambda i: (0, 0)),
        pl.BlockSpec((1, cin), lambda i: (0, 0)),
        pl.BlockSpec((cin, H), lambda i: (0, 0)),
        pl.BlockSpec((1, H), lambda i: (0, 0)),
        pl.BlockSpec((H, OUT), lambda i: (0, 0)),
        pl.BlockSpec((1, OUT), lambda i: (0, 0)),
        pl.BlockSpec((OUT, H), lambda i: (0, 0)),
        pl.BlockSpec((1, H), lambda i: (0, 0)),
    ]
    if last:
        out_spec = pl.BlockSpec((NB, H), lambda i: (i, 0))
        out_shape = jax.ShapeDtypeStruct((N, H), jnp.float32)
    else:
        out_spec = pl.BlockSpec((2, NB, HC), lambda i: (0, i, 0))
        out_shape = jax.ShapeDtypeStruct((2, N, HC), jnp.float32)
    return pl.pallas_call(
        functools.partial(_node_body, first, last),
        grid=(nblk,),
        in_specs=in_specs,
        out_specs=out_spec,
        out_shape=out_shape,
    )(*args)


def _pool_body(h_ref, b_ref, ps_ref, cnt_ref):
    i = pl.program_id(0)
    mask = (lax.broadcasted_iota(jnp.int32, (NUM_GRAPHS, NB), 0)
            == b_ref[0]).astype(jnp.float32)
    ps = jnp.dot(mask, h_ref[...], preferred_element_type=jnp.float32)
    cnt = jnp.sum(mask, axis=1, keepdims=True)

    @pl.when(i == 0)
    def _():
        ps_ref[...] = jnp.zeros_like(ps_ref)
        cnt_ref[...] = jnp.zeros_like(cnt_ref)

    ps_ref[...] += ps
    cnt_ref[...] += cnt


def _pool(h, batch_row):
    return pl.pallas_call(
        _pool_body,
        grid=(N // NB,),
        in_specs=[pl.BlockSpec((NB, H), lambda i: (i, 0)),
                  pl.BlockSpec((1, 1, NB), lambda i: (i, 0, 0))],
        out_specs=[pl.BlockSpec((NUM_GRAPHS, H), lambda i: (0, 0)),
                   pl.BlockSpec((NUM_GRAPHS, 1), lambda i: (0, 0))],
        out_shape=[jax.ShapeDtypeStruct((NUM_GRAPHS, H), jnp.float32),
                   jax.ShapeDtypeStruct((NUM_GRAPHS, 1), jnp.float32)],
    )(h, batch_row)


def _head_body(ps_ref, cnt_ref, w_ref, b_ref, o_ref):
    pooled = ps_ref[...] / jnp.maximum(cnt_ref[...], 1.0)
    o_ref[...] = jax.nn.sigmoid(
        jnp.dot(pooled, w_ref[...], preferred_element_type=jnp.float32)
        + b_ref[...]) * 0.5


def _head(ps, cnt, wfc, bfc):
    return pl.pallas_call(
        _head_body,
        out_shape=jax.ShapeDtypeStruct((NUM_GRAPHS, 1), jnp.float32),
    )(ps, cnt, wfc, bfc.reshape(1, 1))


def kernel(x, edge_index, edge_attr, xA, dosd_distances, batch, params):
    p = params
    src = edge_index[0]
    dst = edge_index[1]

    flat = src * N + dst
    dosd_vals = _dosd_gather(dosd_distances.reshape(N * N), flat)
    dv = dosd_vals.reshape(E, 1)

    batch_row = batch.reshape(N // NB, 1, NB)

    eaw1 = _eaw(1, edge_attr, dv, p)
    aggr1 = _edge_stage(True)(x, eaw1, src, dst)
    eaw2 = _eaw(2, edge_attr, dv, p)
    h2 = _node_mlp(1, aggr1, x, xA, p).reshape(2 * N, HC)
    aggr2 = _edge_stage(False)(h2, eaw2.reshape(2 * E, HC), src, dst)
    eaw3 = _eaw(3, edge_attr, dv, p)
    h3 = _node_mlp(2, aggr2, h2, xA, p).reshape(2 * N, HC)
    aggr3 = _edge_stage(False)(h3, eaw3.reshape(2 * E, HC), src, dst)
    h_final = _node_mlp(3, aggr3, h3, xA, p)
    ps, cnt = _pool(h_final, batch_row)
    return _head(ps, cnt, p["Wfc"], p["bfc"])


# speedup vs baseline: 1.0307x; 1.0307x over previous
import functools

import jax
import jax.numpy as jnp
from jax import lax
from jax.experimental import pallas as pl
from jax.experimental.pallas import tpu as pltpu
from jax.experimental.pallas import tpu_sc as plsc

N = 10000
E = 320000
NNFEAT = 128
H = 256
OUT = 1024
NUM_GRAPHS = 16

NC = 2
NS = 16
NW = NC * NS
L = 16

K = 80
EPW = E // NW
EPS = E // NS
NPS = N // NS
HC = 128
HW = HC // 2
WR = 624

_MESH = plsc.VectorSubcoreMesh(core_axis_name="c", subcore_axis_name="s")


def _dosd_body(tab_hbm, fidx_hbm, out_hbm, fidx0, fidx1, vbuf0, vbuf1,
               isem0, isem1, gsem0, gsem1):
    c = lax.axis_index("c")
    s = lax.axis_index("s")
    wid = s * NC + c
    base0 = wid * EPW
    sets = ((fidx0, vbuf0, isem0, gsem0), (fidx1, vbuf1, isem1, gsem1))

    def issue_idx(j, p):
        fi, _, ise, _ = sets[p]
        pltpu.async_copy(fidx_hbm.at[pl.ds(base0 + j * K, K)], fi, ise)

    def issue_gather(p):
        fi, vb, ise, gse = sets[p]
        pltpu.make_async_copy(fidx_hbm.at[pl.ds(0, K)], fi, ise).wait()
        pltpu.async_copy(tab_hbm.at[fi], vb, gse)

    def drain(j, p):
        fi, vb, _, gse = sets[p]
        pltpu.make_async_copy(tab_hbm.at[fi], vb, gse).wait()
        pltpu.sync_copy(vb, out_hbm.at[pl.ds(base0 + j * K, K)])

    issue_idx(0, 0)
    issue_idx(1, 1)
    issue_gather(0)

    def pair(t, _):
        j = t * 2

        def half(j, p):
            pl.when(j + 1 < EPW // K)(lambda: issue_gather(1 - p))
            drain(j, p)
            pl.when(j + 2 < EPW // K)(lambda: issue_idx(j + 2, p))

        half(j, 0)
        half(j + 1, 1)
        return 0

    lax.fori_loop(0, EPW // (2 * K), pair, 0)
    if (EPW // K) % 2:
        drain(EPW // K - 1, 0)


_dosd_gather = functools.partial(
    pl.kernel,
    out_type=jax.ShapeDtypeStruct((E,), jnp.float32),
    mesh=_MESH,
    scratch_types=[
        pltpu.VMEM((K,), jnp.int32),
        pltpu.VMEM((K,), jnp.int32),
        pltpu.VMEM((K,), jnp.float32),
        pltpu.VMEM((K,), jnp.float32),
        pltpu.SemaphoreType.DMA,
        pltpu.SemaphoreType.DMA,
        pltpu.SemaphoreType.DMA,
        pltpu.SemaphoreType.DMA,
    ],
)(_dosd_body)


def _edge_body(esplit, h_hbm, eaw_hbm, src_hbm, dst_hbm, out_hbm,
               sidx0, didx0, gidx0, dscat0, hbuf0, ebuf0,
               sidx1, didx1, gidx1, dscat1, hbuf1, ebuf1,
               aggr_sh,
               isem0, gsem0, esem0, ssem0, isem1, gsem1, esem1, ssem1):
    c = lax.axis_index("c")
    s = lax.axis_index("s")

    def zrow(r, _):
        for cc in range(HC // L):
            hbuf0[r, pl.ds(cc * L, L)] = jnp.zeros((L,), jnp.float32)
        return 0

    lax.fori_loop(0, K, zrow, 0)

    def zcopy(j, _):
        pltpu.sync_copy(hbuf0, aggr_sh.at[pl.ds(s * NPS + j * K, K)])
        return 0

    lax.fori_loop(0, NPS // K, zcopy, 0)
    pltpu.sync_copy(hbuf0.at[pl.ds(0, NPS - (NPS // K) * K)],
                    aggr_sh.at[pl.ds(s * NPS + (NPS // K) * K,
                                     NPS - (NPS // K) * K)])
    plsc.subcore_barrier()

    if esplit:
        base0 = (c * NS + s) * EPW
        goff = 0
        eoff = 0
        nchunks = EPW // K
    else:
        base0 = s * EPS
        goff = c * N
        eoff = c * E
        nchunks = EPS // K

    sets = ((sidx0, didx0, gidx0, dscat0, hbuf0, ebuf0, isem0, gsem0, esem0,
             ssem0),
            (sidx1, didx1, gidx1, dscat1, hbuf1, ebuf1, isem1, gsem1, esem1,
             ssem1))

    def issue_idx(j, p):
        b = base0 + j * K
        si, di, _, _, _, _, ise, _, _, _ = sets[p]
        pltpu.async_copy(src_hbm.at[pl.ds(b, K)], si, ise)
        pltpu.async_copy(dst_hbm.at[pl.ds(b, K)], di, ise)

    def wait_idx(p):
        si, di, _, _, _, _, ise, _, _, _ = sets[p]
        pltpu.make_async_copy(src_hbm.at[pl.ds(0, K)], si, ise).wait()
        pltpu.make_async_copy(dst_hbm.at[pl.ds(0, K)], di, ise).wait()

    def issue_fetch(j, p):
        b = base0 + j * K
        si, di, gi, dsc, hb, eb, _, gse, ese, _ = sets[p]
        for i in range(K // L):
            sl = pl.ds(i * L, L)
            gi[sl] = si[sl] + goff
            dsc[sl] = di[sl]
        pltpu.async_copy(h_hbm.at[gi], hb, gse)
        pltpu.async_copy(eaw_hbm.at[pl.ds(eoff + b, K)], eb, ese)

    def wait_fetch(p):
        _, _, gi, _, hb, eb, _, gse, ese, _ = sets[p]
        pltpu.make_async_copy(h_hbm.at[gi], hb, gse).wait()
        pltpu.make_async_copy(eaw_hbm.at[pl.ds(0, K)], eb, ese).wait()

    def compute(p):
        hb, eb = sets[p][4], sets[p][5]

        def row(r, _):
            for g in range(HC // 32):
                w = eb[r, pl.ds(g * L, L)]
                lo = plsc.bitcast(w << 16, jnp.float32)
                hi = plsc.bitcast(w & jnp.int32(-65536), jnp.float32)
                sll = pl.ds(g * 32, L)
                slh = pl.ds(g * 32 + L, L)
                hb[r, sll] = jnp.maximum(hb[r, sll] + lo, 0.0)
                hb[r, slh] = jnp.maximum(hb[r, slh] + hi, 0.0)
            return 0

        lax.fori_loop(0, K, row, 0)

    def issue_scatter(p):
        _, _, _, dsc, hb, _, _, _, _, sse = sets[p]
        pltpu.async_copy(hb, aggr_sh.at[dsc], sse, add=True)

    def wait_scatter(p):
        _, _, _, dsc, hb, _, _, _, _, sse = sets[p]
        pltpu.make_async_copy(hb, aggr_sh.at[dsc], sse).wait()

    def body(j, p, static_last=False):
        pbar = 1 - p

        if not static_last:
            @pl.when(j + 1 < nchunks)
            def _():
                pl.when(j >= 1)(lambda: wait_scatter(pbar))
                wait_idx(pbar)
                issue_fetch(j + 1, pbar)
                pl.when(j + 2 < nchunks)(lambda: issue_idx(j + 2, p))

        wait_fetch(p)
        compute(p)
        issue_scatter(p)

    issue_idx(0, 0)
    issue_idx(1, 1)
    wait_idx(0)
    issue_fetch(0, 0)

    def pair(t, _):
        body(t * 2, 0)
        body(t * 2 + 1, 1)
        return 0

    lax.fori_loop(0, nchunks // 2, pair, 0)
    if nchunks % 2:
        body(nchunks - 1, 0, static_last=True)
    wait_scatter(nchunks % 2)
    wait_scatter((nchunks + 1) % 2)
    plsc.subcore_barrier()
    pltpu.sync_copy(aggr_sh.at[pl.ds(s * WR, WR)],
                    out_hbm.at[pl.ds(c * N + s * WR, WR)])

    @pl.when(s == 0)
    def _():
        pltpu.sync_copy(aggr_sh.at[pl.ds(NS * WR, N - NS * WR)],
                        out_hbm.at[pl.ds(c * N + NS * WR, N - NS * WR)])


@functools.cache
def _edge_stage(esplit):
    return functools.partial(
        pl.kernel,
        out_type=jax.ShapeDtypeStruct((2 * N, HC), jnp.float32),
        mesh=_MESH,
        compiler_params=pltpu.CompilerParams(needs_layout_passes=False),
        scratch_types=(
            [pltpu.VMEM((K,), jnp.int32)] * 4
            + [pltpu.VMEM((K, HC), jnp.float32),
               pltpu.VMEM((K, HW), jnp.int32)]
            + [pltpu.VMEM((K,), jnp.int32)] * 4
            + [pltpu.VMEM((K, HC), jnp.float32),
               pltpu.VMEM((K, HW), jnp.int32)]
            + [pltpu.VMEM_SHARED((N, HC), jnp.float32)]
            + [pltpu.SemaphoreType.DMA] * 8
        ),
    )(functools.partial(_edge_body, esplit))


EB = 2000


def _eaw_body(ea_ref, dv_ref, *refs):
    ws = refs[:30]
    outs = refs[30:]
    ea = ea_ref[...]
    dv = dv_ref[...]

    def packed(k):
        wlo, wlol, blo, whi, whil, bhi = ws[6 * k:6 * k + 6]
        ra = (jnp.dot(ea, wlo[...], preferred_element_type=jnp.float32)
              + dv * wlol[...] + blo[...])
        rb = (jnp.dot(ea, whi[...], preferred_element_type=jnp.float32)
              + dv * whil[...] + bhi[...])
        ba = lax.bitcast_convert_type(ra.astype(jnp.bfloat16),
                                      jnp.uint16).astype(jnp.uint32)
        bb = lax.bitcast_convert_type(rb.astype(jnp.bfloat16),
                                      jnp.uint16).astype(jnp.uint32)
        return lax.bitcast_convert_type(ba | (bb << jnp.uint32(16)),
                                        jnp.int32)

    o1, o2, o3 = outs
    o1[...] = packed(0)
    o2[0] = packed(1)
    o2[1] = packed(2)
    o3[0] = packed(3)
    o3[1] = packed(4)


def _eaw_all(edge_attr, dosd_vals, params):
    p = params
    lo_perm = [g * 32 + i for g in range(4) for i in range(16)]
    hi_perm = [g * 32 + 16 + i for g in range(4) for i in range(16)]
    wargs = []
    wspecs = []
    for l, h in ((1, 0), (2, 0), (2, 1), (3, 0), (3, 1)):
        we = p[f"We{l}"]
        be = p[f"be{l}"]
        for perm in (lo_perm, hi_perm):
            cols = jnp.asarray([h * 128 + q for q in perm])
            wp = we[:, cols]
            wargs += [wp[:17], wp[17].reshape(1, HW), be[cols].reshape(1, HW)]
            wspecs += [pl.BlockSpec((17, HW), lambda i: (0, 0)),
                       pl.BlockSpec((1, HW), lambda i: (0, 0)),
                       pl.BlockSpec((1, HW), lambda i: (0, 0))]
    return pl.pallas_call(
        _eaw_body,
        grid=(E // EB,),
        in_specs=[pl.BlockSpec((EB, 17), lambda i: (i, 0)),
                  pl.BlockSpec((EB, 1), lambda i: (i, 0))] + wspecs,
        out_specs=[pl.BlockSpec((EB, HW), lambda i: (i, 0)),
                   pl.BlockSpec((2, EB, HW), lambda i: (0, i, 0)),
                   pl.BlockSpec((2, EB, HW), lambda i: (0, i, 0))],
        out_shape=[jax.ShapeDtypeStruct((E, HW), jnp.int32),
                   jax.ShapeDtypeStruct((2, E, HW), jnp.int32),
                   jax.ShapeDtypeStruct((2, E, HW), jnp.int32)],
    )(edge_attr, dosd_vals, *wargs)


NB = 1000


def _node_body(first, last, alo_ref, ahi_ref, hlo_ref, hhi_ref, xa_ref,
               wg_ref, bg_ref, wa_ref, ba_ref, wb_ref, bb_ref, wf_ref,
               bf_ref, o_ref):
    gt = jnp.dot(xa_ref[...], wg_ref[...],
                 preferred_element_type=jnp.float32) + bg_ref[...]
    if first:
        inp = alo_ref[...] + ahi_ref[...] + hlo_ref[...] + gt
    else:
        inp = jnp.concatenate(
            [alo_ref[...] + hlo_ref[...], ahi_ref[...] + hhi_ref[...]],
            axis=1) + gt
    bf = jnp.bfloat16
    t1 = jax.nn.relu(jnp.dot(inp.astype(bf), wa_ref[...].astype(bf),
                             preferred_element_type=jnp.float32) + ba_ref[...])
    t2 = jax.nn.relu(jnp.dot(t1.astype(bf), wb_ref[...].astype(bf),
                             preferred_element_type=jnp.float32) + bb_ref[...])
    h3 = jnp.dot(t2.astype(bf), wf_ref[...].astype(bf),
                 preferred_element_type=jnp.float32) + bf_ref[...]
    if not last:
        h3 = jax.nn.relu(h3)
        o_ref[0] = h3[:, :HC]
        o_ref[1] = h3[:, HC:]
    else:
        o_ref[...] = h3


def _node_mlp(l, aggr_flat, h_flat, xA, params):
    p = params
    cin = NNFEAT if l == 1 else H
    first = l == 1
    hcin = cin if first else cin // 2
    last = l == 3
    nblk = N // NB
    args = [aggr_flat, aggr_flat, h_flat, h_flat,
            xA.reshape(1, 21), p[f"Wg{l}"], p[f"bg{l}"].reshape(1, cin),
            p[f"W{l}a"], p[f"b{l}a"].reshape(1, H),
            p[f"W{l}b"], p[f"b{l}b"].reshape(1, OUT),
            p[f"Wf{l}"], p[f"bf{l}"].reshape(1, H)]
    in_specs = [
        pl.BlockSpec((NB, HC), lambda i: (i, 0)),
        pl.BlockSpec((NB, HC), lambda i: (i + nblk, 0)),
        pl.BlockSpec((NB, hcin), lambda i: (i, 0)),
        pl.BlockSpec((NB, hcin), lambda i: (i, 0) if first
                     else (i + nblk, 0)),
        pl.BlockSpec((1, 21), lambda i: (0, 0)),
        pl.BlockSpec((21, cin), lambda i: (0, 0)),
        pl.BlockSpec((1, cin), lambda i: (0, 0)),
        pl.BlockSpec((cin, H), lambda i: (0, 0)),
        pl.BlockSpec((1, H), lambda i: (0, 0)),
        pl.BlockSpec((H, OUT), lambda i: (0, 0)),
        pl.BlockSpec((1, OUT), lambda i: (0, 0)),
        pl.BlockSpec((OUT, H), lambda i: (0, 0)),
        pl.BlockSpec((1, H), lambda i: (0, 0)),
    ]
    if last:
        out_spec = pl.BlockSpec((NB, H), lambda i: (i, 0))
        out_shape = jax.ShapeDtypeStruct((N, H), jnp.float32)
    else:
        out_spec = pl.BlockSpec((2, NB, HC), lambda i: (0, i, 0))
        out_shape = jax.ShapeDtypeStruct((2, N, HC), jnp.float32)
    return pl.pallas_call(
        functools.partial(_node_body, first, last),
        grid=(nblk,),
        in_specs=in_specs,
        out_specs=out_spec,
        out_shape=out_shape,
    )(*args)


def _pool_body(h_ref, b_ref, ps_ref, cnt_ref):
    i = pl.program_id(0)
    mask = (lax.broadcasted_iota(jnp.int32, (NUM_GRAPHS, NB), 0)
            == b_ref[0]).astype(jnp.float32)
    ps = jnp.dot(mask, h_ref[...], preferred_element_type=jnp.float32)
    cnt = jnp.sum(mask, axis=1, keepdims=True)

    @pl.when(i == 0)
    def _():
        ps_ref[...] = jnp.zeros_like(ps_ref)
        cnt_ref[...] = jnp.zeros_like(cnt_ref)

    ps_ref[...] += ps
    cnt_ref[...] += cnt


def _pool(h, batch_row):
    return pl.pallas_call(
        _pool_body,
        grid=(N // NB,),
        in_specs=[pl.BlockSpec((NB, H), lambda i: (i, 0)),
                  pl.BlockSpec((1, 1, NB), lambda i: (i, 0, 0))],
        out_specs=[pl.BlockSpec((NUM_GRAPHS, H), lambda i: (0, 0)),
                   pl.BlockSpec((NUM_GRAPHS, 1), lambda i: (0, 0))],
        out_shape=[jax.ShapeDtypeStruct((NUM_GRAPHS, H), jnp.float32),
                   jax.ShapeDtypeStruct((NUM_GRAPHS, 1), jnp.float32)],
    )(h, batch_row)


def _head_body(ps_ref, cnt_ref, w_ref, b_ref, o_ref):
    pooled = ps_ref[...] / jnp.maximum(cnt_ref[...], 1.0)
    o_ref[...] = jax.nn.sigmoid(
        jnp.dot(pooled, w_ref[...], preferred_element_type=jnp.float32)
        + b_ref[...]) * 0.5


def _head(ps, cnt, wfc, bfc):
    return pl.pallas_call(
        _head_body,
        out_shape=jax.ShapeDtypeStruct((NUM_GRAPHS, 1), jnp.float32),
    )(ps, cnt, wfc, bfc.reshape(1, 1))


def kernel(x, edge_index, edge_attr, xA, dosd_distances, batch, params):
    p = params
    src = edge_index[0]
    dst = edge_index[1]

    flat = src * N + dst
    dosd_vals = _dosd_gather(dosd_distances.reshape(N * N), flat)
    dv = dosd_vals.reshape(E, 1)

    batch_row = batch.reshape(N // NB, 1, NB)

    eaw1, eaw2, eaw3 = _eaw_all(edge_attr, dv, p)
    aggr1 = _edge_stage(True)(x, eaw1, src, dst)
    h2 = _node_mlp(1, aggr1, x, xA, p).reshape(2 * N, HC)
    aggr2 = _edge_stage(False)(h2, eaw2.reshape(2 * E, HW), src, dst)
    h3 = _node_mlp(2, aggr2, h2, xA, p).reshape(2 * N, HC)
    aggr3 = _edge_stage(False)(h3, eaw3.reshape(2 * E, HW), src, dst)
    h_final = _node_mlp(3, aggr3, h3, xA, p)
    ps, cnt = _pool(h_final, batch_row)
    return _head(ps, cnt, p["Wfc"], p["bfc"])


# speedup vs baseline: 1.1965x; 1.1608x over previous
import functools

import jax
import jax.numpy as jnp
from jax import lax
from jax.experimental import pallas as pl
from jax.experimental.pallas import tpu as pltpu
from jax.experimental.pallas import tpu_sc as plsc

N = 10000
E = 320000
NNFEAT = 128
H = 256
OUT = 1024
NUM_GRAPHS = 16

NC = 2
NS = 16
NW = NC * NS
L = 16

K = 80
EPW = E // NW
EPS = E // NS
NPS = N // NS
HC = 128
HW = HC // 2
WR = 624

_MESH = plsc.VectorSubcoreMesh(core_axis_name="c", subcore_axis_name="s")


def _dosd_body(tab_hbm, fidx_hbm, out_hbm, fidx0, fidx1, vbuf0, vbuf1,
               isem0, isem1, gsem0, gsem1):
    c = lax.axis_index("c")
    s = lax.axis_index("s")
    wid = s * NC + c
    base0 = wid * EPW
    sets = ((fidx0, vbuf0, isem0, gsem0), (fidx1, vbuf1, isem1, gsem1))

    def issue_idx(j, p):
        fi, _, ise, _ = sets[p]
        pltpu.async_copy(fidx_hbm.at[pl.ds(base0 + j * K, K)], fi, ise)

    def issue_gather(p):
        fi, vb, ise, gse = sets[p]
        pltpu.make_async_copy(fidx_hbm.at[pl.ds(0, K)], fi, ise).wait()
        pltpu.async_copy(tab_hbm.at[fi], vb, gse)

    def drain(j, p):
        fi, vb, _, gse = sets[p]
        pltpu.make_async_copy(tab_hbm.at[fi], vb, gse).wait()
        pltpu.sync_copy(vb, out_hbm.at[pl.ds(base0 + j * K, K)])

    issue_idx(0, 0)
    issue_idx(1, 1)
    issue_gather(0)

    def pair(t, _):
        j = t * 2

        def half(j, p):
            pl.when(j + 1 < EPW // K)(lambda: issue_gather(1 - p))
            drain(j, p)
            pl.when(j + 2 < EPW // K)(lambda: issue_idx(j + 2, p))

        half(j, 0)
        half(j + 1, 1)
        return 0

    lax.fori_loop(0, EPW // (2 * K), pair, 0)
    if (EPW // K) % 2:
        drain(EPW // K - 1, 0)


_dosd_gather = functools.partial(
    pl.kernel,
    out_type=jax.ShapeDtypeStruct((E,), jnp.float32),
    mesh=_MESH,
    scratch_types=[
        pltpu.VMEM((K,), jnp.int32),
        pltpu.VMEM((K,), jnp.int32),
        pltpu.VMEM((K,), jnp.float32),
        pltpu.VMEM((K,), jnp.float32),
        pltpu.SemaphoreType.DMA,
        pltpu.SemaphoreType.DMA,
        pltpu.SemaphoreType.DMA,
        pltpu.SemaphoreType.DMA,
    ],
)(_dosd_body)


def _edge_body(esplit, h_hbm, eaw_hbm, src_hbm, dst_hbm, out_hbm,
               sidx0, didx0, gidx0, dscat0, hbuf0, ebuf0,
               sidx1, didx1, gidx1, dscat1, hbuf1, ebuf1,
               aggr_sh,
               isem0, gsem0, esem0, ssem0, isem1, gsem1, esem1, ssem1):
    c = lax.axis_index("c")
    s = lax.axis_index("s")

    def zrow(r, _):
        for cc in range(HC // L):
            hbuf0[r, pl.ds(cc * L, L)] = jnp.zeros((L,), jnp.float32)
        return 0

    lax.fori_loop(0, K, zrow, 0)

    def zcopy(j, _):
        pltpu.sync_copy(hbuf0, aggr_sh.at[pl.ds(s * NPS + j * K, K)])
        return 0

    lax.fori_loop(0, NPS // K, zcopy, 0)
    pltpu.sync_copy(hbuf0.at[pl.ds(0, NPS - (NPS // K) * K)],
                    aggr_sh.at[pl.ds(s * NPS + (NPS // K) * K,
                                     NPS - (NPS // K) * K)])
    plsc.subcore_barrier()

    if esplit:
        base0 = (c * NS + s) * EPW
        goff = 0
        eoff = 0
        nchunks = EPW // K
    else:
        base0 = s * EPS
        goff = c * N
        eoff = c * E
        nchunks = EPS // K

    sets = ((sidx0, didx0, gidx0, dscat0, hbuf0, ebuf0, isem0, gsem0, esem0,
             ssem0),
            (sidx1, didx1, gidx1, dscat1, hbuf1, ebuf1, isem1, gsem1, esem1,
             ssem1))

    def issue_idx(j, p):
        b = base0 + j * K
        si, di, _, _, _, _, ise, _, _, _ = sets[p]
        pltpu.async_copy(src_hbm.at[pl.ds(b, K)], si, ise)
        pltpu.async_copy(dst_hbm.at[pl.ds(b, K)], di, ise)

    def wait_idx(p):
        si, di, _, _, _, _, ise, _, _, _ = sets[p]
        pltpu.make_async_copy(src_hbm.at[pl.ds(0, K)], si, ise).wait()
        pltpu.make_async_copy(dst_hbm.at[pl.ds(0, K)], di, ise).wait()

    def issue_fetch(j, p):
        b = base0 + j * K
        si, di, gi, dsc, hb, eb, _, gse, ese, _ = sets[p]
        for i in range(K // L):
            sl = pl.ds(i * L, L)
            gi[sl] = si[sl] + goff
            dsc[sl] = di[sl]
        pltpu.async_copy(h_hbm.at[gi], hb, gse)
        pltpu.async_copy(eaw_hbm.at[pl.ds(eoff + b, K)], eb, ese)

    def wait_fetch(p):
        _, _, gi, _, hb, eb, _, gse, ese, _ = sets[p]
        pltpu.make_async_copy(h_hbm.at[gi], hb, gse).wait()
        pltpu.make_async_copy(eaw_hbm.at[pl.ds(0, K)], eb, ese).wait()

    def compute(p):
        hb, eb = sets[p][4], sets[p][5]

        def row(r, _):
            for g in range(HC // 32):
                w = eb[r, pl.ds(g * L, L)]
                lo = plsc.bitcast(w << 16, jnp.float32)
                hi = plsc.bitcast(w & jnp.int32(-65536), jnp.float32)
                sll = pl.ds(g * 32, L)
                slh = pl.ds(g * 32 + L, L)
                hb[r, sll] = jnp.maximum(hb[r, sll] + lo, 0.0)
                hb[r, slh] = jnp.maximum(hb[r, slh] + hi, 0.0)
            return 0

        lax.fori_loop(0, K, row, 0)

    def issue_scatter(p):
        _, _, _, dsc, hb, _, _, _, _, sse = sets[p]
        pltpu.async_copy(hb, aggr_sh.at[dsc], sse, add=True)

    def wait_scatter(p):
        _, _, _, dsc, hb, _, _, _, _, sse = sets[p]
        pltpu.make_async_copy(hb, aggr_sh.at[dsc], sse).wait()

    def body(j, p, static_last=False):
        pbar = 1 - p

        if not static_last:
            @pl.when(j + 1 < nchunks)
            def _():
                pl.when(j >= 1)(lambda: wait_scatter(pbar))
                wait_idx(pbar)
                issue_fetch(j + 1, pbar)
                pl.when(j + 2 < nchunks)(lambda: issue_idx(j + 2, p))

        wait_fetch(p)
        compute(p)
        issue_scatter(p)

    issue_idx(0, 0)
    issue_idx(1, 1)
    wait_idx(0)
    issue_fetch(0, 0)

    def pair(t, _):
        body(t * 2, 0)
        body(t * 2 + 1, 1)
        return 0

    lax.fori_loop(0, nchunks // 2, pair, 0)
    if nchunks % 2:
        body(nchunks - 1, 0, static_last=True)
    wait_scatter(nchunks % 2)
    wait_scatter((nchunks + 1) % 2)
    plsc.subcore_barrier()
    pltpu.sync_copy(aggr_sh.at[pl.ds(s * WR, WR)],
                    out_hbm.at[pl.ds(c * N + s * WR, WR)])

    @pl.when(s == 0)
    def _():
        pltpu.sync_copy(aggr_sh.at[pl.ds(NS * WR, N - NS * WR)],
                        out_hbm.at[pl.ds(c * N + NS * WR, N - NS * WR)])


@functools.cache
def _edge_stage(esplit):
    return functools.partial(
        pl.kernel,
        out_type=jax.ShapeDtypeStruct((2 * N, HC), jnp.float32),
        mesh=_MESH,
        compiler_params=pltpu.CompilerParams(needs_layout_passes=False),
        scratch_types=(
            [pltpu.VMEM((K,), jnp.int32)] * 4
            + [pltpu.VMEM((K, HC), jnp.float32),
               pltpu.VMEM((K, HW), jnp.int32)]
            + [pltpu.VMEM((K,), jnp.int32)] * 4
            + [pltpu.VMEM((K, HC), jnp.float32),
               pltpu.VMEM((K, HW), jnp.int32)]
            + [pltpu.VMEM_SHARED((N, HC), jnp.float32)]
            + [pltpu.SemaphoreType.DMA] * 8
        ),
    )(functools.partial(_edge_body, esplit))


EB = 2560


def _eaw_body(ea_ref, *refs):
    ws = refs[:20]
    outs = refs[20:]
    ea = ea_ref[...]
    dn = (((0,), (0,)), ((), ()))

    def packed(k):
        wlo, blo, whi, bhi = ws[4 * k:4 * k + 4]
        ra = lax.dot_general(ea, wlo[...], dn,
                             preferred_element_type=jnp.float32) + blo[...]
        rb = lax.dot_general(ea, whi[...], dn,
                             preferred_element_type=jnp.float32) + bhi[...]
        ba = lax.bitcast_convert_type(ra.astype(jnp.bfloat16),
                                      jnp.uint16).astype(jnp.uint32)
        bb = lax.bitcast_convert_type(rb.astype(jnp.bfloat16),
                                      jnp.uint16).astype(jnp.uint32)
        return lax.bitcast_convert_type(ba | (bb << jnp.uint32(16)),
                                        jnp.int32)

    o1, o2, o3 = outs
    o1[...] = packed(0)
    o2[0] = packed(1)
    o2[1] = packed(2)
    o3[0] = packed(3)
    o3[1] = packed(4)


def _eaw_all(ea18_t, params):
    p = params
    lo_perm = [g * 32 + i for g in range(4) for i in range(16)]
    hi_perm = [g * 32 + 16 + i for g in range(4) for i in range(16)]
    wargs = []
    wspecs = []
    for l, h in ((1, 0), (2, 0), (2, 1), (3, 0), (3, 1)):
        we = p[f"We{l}"]
        be = p[f"be{l}"]
        for perm in (lo_perm, hi_perm):
            cols = jnp.asarray([h * 128 + q for q in perm])
            wargs += [we[:, cols], be[cols].reshape(1, HW)]
            wspecs += [pl.BlockSpec((18, HW), lambda i: (0, 0)),
                       pl.BlockSpec((1, HW), lambda i: (0, 0))]
    return pl.pallas_call(
        _eaw_body,
        grid=(E // EB,),
        in_specs=[pl.BlockSpec((18, EB), lambda i: (0, i))] + wspecs,
        out_specs=[pl.BlockSpec((EB, HW), lambda i: (i, 0)),
                   pl.BlockSpec((2, EB, HW), lambda i: (0, i, 0)),
                   pl.BlockSpec((2, EB, HW), lambda i: (0, i, 0))],
        out_shape=[jax.ShapeDtypeStruct((E, HW), jnp.int32),
                   jax.ShapeDtypeStruct((2, E, HW), jnp.int32),
                   jax.ShapeDtypeStruct((2, E, HW), jnp.int32)],
    )(ea18_t, *wargs)


NB = 1000


def _node_body(first, last, alo_ref, ahi_ref, hlo_ref, hhi_ref, xa_ref,
               wg_ref, bg_ref, wa_ref, ba_ref, wb_ref, bb_ref, wf_ref,
               bf_ref, *rest):
    if last:
        b_ref, ps_ref, cnt_ref = rest
    else:
        o_ref, = rest
    gt = jnp.dot(xa_ref[...], wg_ref[...],
                 preferred_element_type=jnp.float32) + bg_ref[...]
    if first:
        inp = alo_ref[...] + ahi_ref[...] + hlo_ref[...] + gt
    else:
        inp = jnp.concatenate(
            [alo_ref[...] + hlo_ref[...], ahi_ref[...] + hhi_ref[...]],
            axis=1) + gt
    bf = jnp.bfloat16
    t1 = jax.nn.relu(jnp.dot(inp.astype(bf), wa_ref[...].astype(bf),
                             preferred_element_type=jnp.float32) + ba_ref[...])
    t2 = jax.nn.relu(jnp.dot(t1.astype(bf), wb_ref[...].astype(bf),
                             preferred_element_type=jnp.float32) + bb_ref[...])
    h3 = jnp.dot(t2.astype(bf), wf_ref[...].astype(bf),
                 preferred_element_type=jnp.float32) + bf_ref[...]
    if not last:
        h3 = jax.nn.relu(h3)
        o_ref[0] = h3[:, :HC]
        o_ref[1] = h3[:, HC:]
    else:
        i = pl.program_id(0)
        mask = (lax.broadcasted_iota(jnp.int32, (NUM_GRAPHS, NB), 0)
                == b_ref[0]).astype(jnp.float32)
        ps = jnp.dot(mask, h3, preferred_element_type=jnp.float32)
        cnt = jnp.sum(mask, axis=1, keepdims=True)

        @pl.when(i == 0)
        def _():
            ps_ref[...] = jnp.zeros_like(ps_ref)
            cnt_ref[...] = jnp.zeros_like(cnt_ref)

        ps_ref[...] += ps
        cnt_ref[...] += cnt


def _node_mlp(l, aggr_flat, h_flat, xA, params, batch_row=None):
    p = params
    cin = NNFEAT if l == 1 else H
    first = l == 1
    hcin = cin if first else cin // 2
    last = l == 3
    nblk = N // NB
    args = [aggr_flat, aggr_flat, h_flat, h_flat,
            xA.reshape(1, 21), p[f"Wg{l}"], p[f"bg{l}"].reshape(1, cin),
            p[f"W{l}a"], p[f"b{l}a"].reshape(1, H),
            p[f"W{l}b"], p[f"b{l}b"].reshape(1, OUT),
            p[f"Wf{l}"], p[f"bf{l}"].reshape(1, H)]
    in_specs = [
        pl.BlockSpec((NB, HC), lambda i: (i, 0)),
        pl.BlockSpec((NB, HC), lambda i: (i + nblk, 0)),
        pl.BlockSpec((NB, hcin), lambda i: (i, 0)),
        pl.BlockSpec((NB, hcin), lambda i: (i, 0) if first
                     else (i + nblk, 0)),
        pl.BlockSpec((1, 21), lambda i: (0, 0)),
        pl.BlockSpec((21, cin), lambda i: (0, 0)),
        pl.BlockSpec((1, cin), lambda i: (0, 0)),
        pl.BlockSpec((cin, H), lambda i: (0, 0)),
        pl.BlockSpec((1, H), lambda i: (0, 0)),
        pl.BlockSpec((H, OUT), lambda i: (0, 0)),
        pl.BlockSpec((1, OUT), lambda i: (0, 0)),
        pl.BlockSpec((OUT, H), lambda i: (0, 0)),
        pl.BlockSpec((1, H), lambda i: (0, 0)),
    ]
    if last:
        args.append(batch_row)
        in_specs.append(pl.BlockSpec((1, 1, NB), lambda i: (i, 0, 0)))
        out_spec = [pl.BlockSpec((NUM_GRAPHS, H), lambda i: (0, 0)),
                    pl.BlockSpec((NUM_GRAPHS, 1), lambda i: (0, 0))]
        out_shape = [jax.ShapeDtypeStruct((NUM_GRAPHS, H), jnp.float32),
                     jax.ShapeDtypeStruct((NUM_GRAPHS, 1), jnp.float32)]
    else:
        out_spec = pl.BlockSpec((2, NB, HC), lambda i: (0, i, 0))
        out_shape = jax.ShapeDtypeStruct((2, N, HC), jnp.float32)
    return pl.pallas_call(
        functools.partial(_node_body, first, last),
        grid=(nblk,),
        in_specs=in_specs,
        out_specs=out_spec,
        out_shape=out_shape,
    )(*args)


def _head_body(ps_ref, cnt_ref, w_ref, b_ref, o_ref):
    pooled = ps_ref[...] / jnp.maximum(cnt_ref[...], 1.0)
    o_ref[...] = jax.nn.sigmoid(
        jnp.dot(pooled, w_ref[...], preferred_element_type=jnp.float32)
        + b_ref[...]) * 0.5


def _head(ps, cnt, wfc, bfc):
    return pl.pallas_call(
        _head_body,
        out_shape=jax.ShapeDtypeStruct((NUM_GRAPHS, 1), jnp.float32),
    )(ps, cnt, wfc, bfc.reshape(1, 1))


def kernel(x, edge_index, edge_attr, xA, dosd_distances, batch, params):
    p = params
    src = edge_index[0]
    dst = edge_index[1]

    flat = src * N + dst
    dosd_vals = _dosd_gather(dosd_distances.reshape(N * N), flat)
    ea18_t = jnp.concatenate([edge_attr.T, dosd_vals.reshape(1, E)], axis=0)

    batch_row = batch.reshape(N // NB, 1, NB)

    eaw1, eaw2, eaw3 = _eaw_all(ea18_t, p)
    aggr1 = _edge_stage(True)(x, eaw1, src, dst)
    h2 = _node_mlp(1, aggr1, x, xA, p).reshape(2 * N, HC)
    aggr2 = _edge_stage(False)(h2, eaw2.reshape(2 * E, HW), src, dst)
    h3 = _node_mlp(2, aggr2, h2, xA, p).reshape(2 * N, HC)
    aggr3 = _edge_stage(False)(h3, eaw3.reshape(2 * E, HW), src, dst)
    ps, cnt = _node_mlp(3, aggr3, h3, xA, p, batch_row)
    return _head(ps, cnt, p["Wfc"], p["bfc"])


# speedup vs baseline: 1.2012x; 1.0039x over previous
import functools

import jax
import jax.numpy as jnp
from jax import lax
from jax.experimental import pallas as pl
from jax.experimental.pallas import tpu as pltpu
from jax.experimental.pallas import tpu_sc as plsc

N = 10000
E = 320000
NNFEAT = 128
H = 256
OUT = 1024
NUM_GRAPHS = 16

NC = 2
NS = 16
NW = NC * NS
L = 16

K = 80
EPW = E // NW
EPS = E // NS
NPS = N // NS
HC = 128
HW = HC // 2
WR = 624

_MESH = plsc.VectorSubcoreMesh(core_axis_name="c", subcore_axis_name="s")


def _dosd_body(tab_hbm, fidx_hbm, out_hbm, fidx0, fidx1, vbuf0, vbuf1,
               isem0, isem1, gsem0, gsem1):
    c = lax.axis_index("c")
    s = lax.axis_index("s")
    wid = s * NC + c
    base0 = wid * EPW
    sets = ((fidx0, vbuf0, isem0, gsem0), (fidx1, vbuf1, isem1, gsem1))

    def issue_idx(j, p):
        fi, _, ise, _ = sets[p]
        pltpu.async_copy(fidx_hbm.at[pl.ds(base0 + j * K, K)], fi, ise)

    def issue_gather(p):
        fi, vb, ise, gse = sets[p]
        pltpu.make_async_copy(fidx_hbm.at[pl.ds(0, K)], fi, ise).wait()
        pltpu.async_copy(tab_hbm.at[fi], vb, gse)

    def drain(j, p):
        fi, vb, _, gse = sets[p]
        pltpu.make_async_copy(tab_hbm.at[fi], vb, gse).wait()
        pltpu.sync_copy(vb, out_hbm.at[pl.ds(base0 + j * K, K)])

    issue_idx(0, 0)
    issue_idx(1, 1)
    issue_gather(0)

    def pair(t, _):
        j = t * 2

        def half(j, p):
            pl.when(j + 1 < EPW // K)(lambda: issue_gather(1 - p))
            drain(j, p)
            pl.when(j + 2 < EPW // K)(lambda: issue_idx(j + 2, p))

        half(j, 0)
        half(j + 1, 1)
        return 0

    lax.fori_loop(0, EPW // (2 * K), pair, 0)
    if (EPW // K) % 2:
        drain(EPW // K - 1, 0)


_dosd_gather = functools.partial(
    pl.kernel,
    out_type=jax.ShapeDtypeStruct((E,), jnp.float32),
    mesh=_MESH,
    scratch_types=[
        pltpu.VMEM((K,), jnp.int32),
        pltpu.VMEM((K,), jnp.int32),
        pltpu.VMEM((K,), jnp.float32),
        pltpu.VMEM((K,), jnp.float32),
        pltpu.SemaphoreType.DMA,
        pltpu.SemaphoreType.DMA,
        pltpu.SemaphoreType.DMA,
        pltpu.SemaphoreType.DMA,
    ],
)(_dosd_body)


def _edge_body(esplit, h_hbm, eaw_hbm, src_hbm, dst_hbm, out_hbm,
               sidx0, didx0, gidx0, dscat0, hbuf0, ebuf0,
               sidx1, didx1, gidx1, dscat1, hbuf1, ebuf1,
               aggr_sh,
               isem0, gsem0, esem0, ssem0, isem1, gsem1, esem1, ssem1):
    c = lax.axis_index("c")
    s = lax.axis_index("s")

    def zrow(r, _):
        for cc in range(HC // L):
            hbuf0[r, pl.ds(cc * L, L)] = jnp.zeros((L,), jnp.float32)
        return 0

    lax.fori_loop(0, K, zrow, 0)

    def zcopy(j, _):
        pltpu.sync_copy(hbuf0, aggr_sh.at[pl.ds(s * NPS + j * K, K)])
        return 0

    lax.fori_loop(0, NPS // K, zcopy, 0)
    pltpu.sync_copy(hbuf0.at[pl.ds(0, NPS - (NPS // K) * K)],
                    aggr_sh.at[pl.ds(s * NPS + (NPS // K) * K,
                                     NPS - (NPS // K) * K)])
    plsc.subcore_barrier()

    if esplit:
        base0 = (c * NS + s) * EPW
        goff = 0
        eoff = 0
        nchunks = EPW // K
    else:
        base0 = s * EPS
        goff = c * N
        eoff = c * E
        nchunks = EPS // K

    sets = ((sidx0, didx0, gidx0, dscat0, hbuf0, ebuf0, isem0, gsem0, esem0,
             ssem0),
            (sidx1, didx1, gidx1, dscat1, hbuf1, ebuf1, isem1, gsem1, esem1,
             ssem1))

    def issue_idx(j, p):
        b = base0 + j * K
        si, di, _, _, _, _, ise, _, _, _ = sets[p]
        pltpu.async_copy(src_hbm.at[pl.ds(b, K)], si, ise)
        pltpu.async_copy(dst_hbm.at[pl.ds(b, K)], di, ise)

    def wait_idx(p):
        si, di, _, _, _, _, ise, _, _, _ = sets[p]
        pltpu.make_async_copy(src_hbm.at[pl.ds(0, K)], si, ise).wait()
        pltpu.make_async_copy(dst_hbm.at[pl.ds(0, K)], di, ise).wait()

    def issue_fetch(j, p):
        b = base0 + j * K
        si, di, gi, dsc, hb, eb, _, gse, ese, _ = sets[p]
        for i in range(K // L):
            sl = pl.ds(i * L, L)
            gi[sl] = si[sl] + goff
            dsc[sl] = di[sl]
        pltpu.async_copy(h_hbm.at[gi], hb, gse)
        pltpu.async_copy(eaw_hbm.at[pl.ds(eoff + b, K)], eb, ese)

    def wait_fetch(p):
        _, _, gi, _, hb, eb, _, gse, ese, _ = sets[p]
        pltpu.make_async_copy(h_hbm.at[gi], hb, gse).wait()
        pltpu.make_async_copy(eaw_hbm.at[pl.ds(0, K)], eb, ese).wait()

    def compute(p):
        hb, eb = sets[p][4], sets[p][5]

        def row(r, _):
            for g in range(HC // 32):
                w = eb[r, pl.ds(g * L, L)]
                lo = plsc.bitcast(w << 16, jnp.float32)
                hi = plsc.bitcast(w & jnp.int32(-65536), jnp.float32)
                sll = pl.ds(g * 32, L)
                slh = pl.ds(g * 32 + L, L)
                hb[r, sll] = jnp.maximum(hb[r, sll] + lo, 0.0)
                hb[r, slh] = jnp.maximum(hb[r, slh] + hi, 0.0)
            return 0

        lax.fori_loop(0, K, row, 0)

    def issue_scatter(p):
        _, _, _, dsc, hb, _, _, _, _, sse = sets[p]
        pltpu.async_copy(hb, aggr_sh.at[dsc], sse, add=True)

    def wait_scatter(p):
        _, _, _, dsc, hb, _, _, _, _, sse = sets[p]
        pltpu.make_async_copy(hb, aggr_sh.at[dsc], sse).wait()

    def body(j, p, static_last=False):
        pbar = 1 - p

        if not static_last:
            @pl.when(j + 1 < nchunks)
            def _():
                pl.when(j >= 1)(lambda: wait_scatter(pbar))
                wait_idx(pbar)
                issue_fetch(j + 1, pbar)
                pl.when(j + 2 < nchunks)(lambda: issue_idx(j + 2, p))

        wait_fetch(p)
        compute(p)
        issue_scatter(p)

    issue_idx(0, 0)
    issue_idx(1, 1)
    wait_idx(0)
    issue_fetch(0, 0)

    def pair(t, _):
        body(t * 2, 0)
        body(t * 2 + 1, 1)
        return 0

    lax.fori_loop(0, nchunks // 2, pair, 0)
    if nchunks % 2:
        body(nchunks - 1, 0, static_last=True)
    wait_scatter(nchunks % 2)
    wait_scatter((nchunks + 1) % 2)
    plsc.subcore_barrier()
    pltpu.sync_copy(aggr_sh.at[pl.ds(s * WR, WR)],
                    out_hbm.at[pl.ds(c * N + s * WR, WR)])

    @pl.when(s == 0)
    def _():
        pltpu.sync_copy(aggr_sh.at[pl.ds(NS * WR, N - NS * WR)],
                        out_hbm.at[pl.ds(c * N + NS * WR, N - NS * WR)])


@functools.cache
def _edge_stage(esplit):
    return functools.partial(
        pl.kernel,
        out_type=jax.ShapeDtypeStruct((2 * N, HC), jnp.float32),
        mesh=_MESH,
        compiler_params=pltpu.CompilerParams(needs_layout_passes=False),
        scratch_types=(
            [pltpu.VMEM((K,), jnp.int32)] * 4
            + [pltpu.VMEM((K, HC), jnp.float32),
               pltpu.VMEM((K, HW), jnp.int32)]
            + [pltpu.VMEM((K,), jnp.int32)] * 4
            + [pltpu.VMEM((K, HC), jnp.float32),
               pltpu.VMEM((K, HW), jnp.int32)]
            + [pltpu.VMEM_SHARED((N, HC), jnp.float32)]
            + [pltpu.SemaphoreType.DMA] * 8
        ),
    )(functools.partial(_edge_body, esplit))


EB = 2560


def _eaw_body(ea_ref, *refs):
    ws = refs[:20]
    outs = refs[20:]
    ea = ea_ref[...]
    dn = (((0,), (0,)), ((), ()))

    def packed(k):
        wlo, blo, whi, bhi = ws[4 * k:4 * k + 4]
        ra = lax.dot_general(ea, wlo[...], dn,
                             preferred_element_type=jnp.float32) + blo[...]
        rb = lax.dot_general(ea, whi[...], dn,
                             preferred_element_type=jnp.float32) + bhi[...]
        ba = lax.bitcast_convert_type(ra.astype(jnp.bfloat16),
                                      jnp.uint16).astype(jnp.uint32)
        bb = lax.bitcast_convert_type(rb.astype(jnp.bfloat16),
                                      jnp.uint16).astype(jnp.uint32)
        return lax.bitcast_convert_type(ba | (bb << jnp.uint32(16)),
                                        jnp.int32)

    o1, o2, o3 = outs
    o1[...] = packed(0)
    o2[0] = packed(1)
    o2[1] = packed(2)
    o3[0] = packed(3)
    o3[1] = packed(4)


def _eaw_all(ea18_t, params):
    p = params
    lo_perm = [g * 32 + i for g in range(4) for i in range(16)]
    hi_perm = [g * 32 + 16 + i for g in range(4) for i in range(16)]
    wargs = []
    wspecs = []
    for l, h in ((1, 0), (2, 0), (2, 1), (3, 0), (3, 1)):
        we = p[f"We{l}"]
        be = p[f"be{l}"]
        for perm in (lo_perm, hi_perm):
            cols = jnp.asarray([h * 128 + q for q in perm])
            wargs += [we[:, cols], be[cols].reshape(1, HW)]
            wspecs += [pl.BlockSpec((18, HW), lambda i: (0, 0)),
                       pl.BlockSpec((1, HW), lambda i: (0, 0))]
    return pl.pallas_call(
        _eaw_body,
        grid=(E // EB,),
        in_specs=[pl.BlockSpec((18, EB), lambda i: (0, i))] + wspecs,
        out_specs=[pl.BlockSpec((EB, HW), lambda i: (i, 0)),
                   pl.BlockSpec((2, EB, HW), lambda i: (0, i, 0)),
                   pl.BlockSpec((2, EB, HW), lambda i: (0, i, 0))],
        out_shape=[jax.ShapeDtypeStruct((E, HW), jnp.int32),
                   jax.ShapeDtypeStruct((2, E, HW), jnp.int32),
                   jax.ShapeDtypeStruct((2, E, HW), jnp.int32)],
    )(ea18_t, *wargs)


NB = 2000


def _node_body(first, last, alo_ref, ahi_ref, hlo_ref, hhi_ref, xa_ref,
               wg_ref, bg_ref, wa_ref, ba_ref, wb_ref, bb_ref, wf_ref,
               bf_ref, *rest):
    if last:
        b_ref, ps_ref, cnt_ref = rest
    else:
        o_ref, = rest
    gt = jnp.dot(xa_ref[...], wg_ref[...],
                 preferred_element_type=jnp.float32) + bg_ref[...]
    if first:
        inp = alo_ref[...] + ahi_ref[...] + hlo_ref[...] + gt
    else:
        inp = jnp.concatenate(
            [alo_ref[...] + hlo_ref[...], ahi_ref[...] + hhi_ref[...]],
            axis=1) + gt
    bf = jnp.bfloat16
    t1 = jax.nn.relu(jnp.dot(inp.astype(bf), wa_ref[...].astype(bf),
                             preferred_element_type=jnp.float32) + ba_ref[...])
    t2 = jax.nn.relu(jnp.dot(t1.astype(bf), wb_ref[...].astype(bf),
                             preferred_element_type=jnp.float32) + bb_ref[...])
    h3 = jnp.dot(t2.astype(bf), wf_ref[...].astype(bf),
                 preferred_element_type=jnp.float32) + bf_ref[...]
    if not last:
        h3 = jax.nn.relu(h3)
        o_ref[0] = h3[:, :HC]
        o_ref[1] = h3[:, HC:]
    else:
        i = pl.program_id(0)
        mask = (lax.broadcasted_iota(jnp.int32, (NUM_GRAPHS, NB), 0)
                == b_ref[0]).astype(jnp.float32)
        ps = jnp.dot(mask, h3, preferred_element_type=jnp.float32)
        cnt = jnp.sum(mask, axis=1, keepdims=True)

        @pl.when(i == 0)
        def _():
            ps_ref[...] = jnp.zeros_like(ps_ref)
            cnt_ref[...] = jnp.zeros_like(cnt_ref)

        ps_ref[...] += ps
        cnt_ref[...] += cnt


def _node_mlp(l, aggr_flat, h_flat, xA, params, batch_row=None):
    p = params
    cin = NNFEAT if l == 1 else H
    first = l == 1
    hcin = cin if first else cin // 2
    last = l == 3
    nblk = N // NB
    args = [aggr_flat, aggr_flat, h_flat, h_flat,
            xA.reshape(1, 21), p[f"Wg{l}"], p[f"bg{l}"].reshape(1, cin),
            p[f"W{l}a"], p[f"b{l}a"].reshape(1, H),
            p[f"W{l}b"], p[f"b{l}b"].reshape(1, OUT),
            p[f"Wf{l}"], p[f"bf{l}"].reshape(1, H)]
    in_specs = [
        pl.BlockSpec((NB, HC), lambda i: (i, 0)),
        pl.BlockSpec((NB, HC), lambda i: (i + nblk, 0)),
        pl.BlockSpec((NB, hcin), lambda i: (i, 0)),
        pl.BlockSpec((NB, hcin), lambda i: (i, 0) if first
                     else (i + nblk, 0)),
        pl.BlockSpec((1, 21), lambda i: (0, 0)),
        pl.BlockSpec((21, cin), lambda i: (0, 0)),
        pl.BlockSpec((1, cin), lambda i: (0, 0)),
        pl.BlockSpec((cin, H), lambda i: (0, 0)),
        pl.BlockSpec((1, H), lambda i: (0, 0)),
        pl.BlockSpec((H, OUT), lambda i: (0, 0)),
        pl.BlockSpec((1, OUT), lambda i: (0, 0)),
        pl.BlockSpec((OUT, H), lambda i: (0, 0)),
        pl.BlockSpec((1, H), lambda i: (0, 0)),
    ]
    if last:
        args.append(batch_row)
        in_specs.append(pl.BlockSpec((1, 1, NB), lambda i: (i, 0, 0)))
        out_spec = [pl.BlockSpec((NUM_GRAPHS, H), lambda i: (0, 0)),
                    pl.BlockSpec((NUM_GRAPHS, 1), lambda i: (0, 0))]
        out_shape = [jax.ShapeDtypeStruct((NUM_GRAPHS, H), jnp.float32),
                     jax.ShapeDtypeStruct((NUM_GRAPHS, 1), jnp.float32)]
    else:
        out_spec = pl.BlockSpec((2, NB, HC), lambda i: (0, i, 0))
        out_shape = jax.ShapeDtypeStruct((2, N, HC), jnp.float32)
    return pl.pallas_call(
        functools.partial(_node_body, first, last),
        grid=(nblk,),
        in_specs=in_specs,
        out_specs=out_spec,
        out_shape=out_shape,
    )(*args)


def _head_body(ps_ref, cnt_ref, w_ref, b_ref, o_ref):
    pooled = ps_ref[...] / jnp.maximum(cnt_ref[...], 1.0)
    o_ref[...] = jax.nn.sigmoid(
        jnp.dot(pooled, w_ref[...], preferred_element_type=jnp.float32)
        + b_ref[...]) * 0.5


def _head(ps, cnt, wfc, bfc):
    return pl.pallas_call(
        _head_body,
        out_shape=jax.ShapeDtypeStruct((NUM_GRAPHS, 1), jnp.float32),
    )(ps, cnt, wfc, bfc.reshape(1, 1))


def kernel(x, edge_index, edge_attr, xA, dosd_distances, batch, params):
    p = params
    src = edge_index[0]
    dst = edge_index[1]

    flat = src * N + dst
    dosd_vals = _dosd_gather(dosd_distances.reshape(N * N), flat)
    ea18_t = jnp.concatenate([edge_attr.T, dosd_vals.reshape(1, E)], axis=0)

    batch_row = batch.reshape(N // NB, 1, NB)

    eaw1, eaw2, eaw3 = _eaw_all(ea18_t, p)
    aggr1 = _edge_stage(True)(x, eaw1, src, dst)
    h2 = _node_mlp(1, aggr1, x, xA, p).reshape(2 * N, HC)
    aggr2 = _edge_stage(False)(h2, eaw2.reshape(2 * E, HW), src, dst)
    h3 = _node_mlp(2, aggr2, h2, xA, p).reshape(2 * N, HC)
    aggr3 = _edge_stage(False)(h3, eaw3.reshape(2 * E, HW), src, dst)
    ps, cnt = _node_mlp(3, aggr3, h3, xA, p, batch_row)
    return _head(ps, cnt, p["Wfc"], p["bfc"])
